# Initial kernel scaffold; baseline (speedup 1.0000x reference)
import jax
import jax.numpy as jnp
from jax.experimental import pallas as pl


def kernel(x, dictionary, lin_w, lin_b, bn_gamma, bn_beta, bn_mean, bn_var, commitment_cost):
    raise NotImplementedError("write your pallas kernel here")


# speedup vs baseline: 17.8769x; 17.8769x over previous
import jax
import jax.numpy as jnp
from jax.experimental import pallas as pl

SPARSITY = 8
EPS = 1e-08
BN_EPS = 1e-05
NEG_BIG = -1e30
TN = 512


def _k1_body(xf_ref, w_ref, d_ref, b_ref, logits_ref, idx_ref, m_ref, s_ref):
    i = pl.program_id(0)
    xf = xf_ref[...]
    w = w_ref[...]
    d = d_ref[...]
    logits = jax.lax.dot_general(xf, w, (((1,), (1,)), ((), ())),
                                 preferred_element_type=jnp.float32) + b_ref[...]
    logits_ref[...] = logits

    tmax = jnp.max(logits, axis=0, keepdims=True)

    @pl.when(i == 0)
    def _():
        m_ref[...] = tmax
        s_ref[...] = jnp.sum(jnp.exp(logits - tmax), axis=0, keepdims=True)

    @pl.when(i > 0)
    def _():
        m_old = m_ref[...]
        m_new = jnp.maximum(m_old, tmax)
        s_ref[...] = (s_ref[...] * jnp.exp(m_old - m_new)
                      + jnp.sum(jnp.exp(logits - m_new), axis=0, keepdims=True))
        m_ref[...] = m_new

    d2 = jnp.sum(d ** 2, axis=1)[None, :]
    x2 = jnp.sum(xf ** 2, axis=1, keepdims=True)
    xd = jax.lax.dot_general(xf, d, (((1,), (1,)), ((), ())),
                             preferred_element_type=jnp.float32)
    scores = -(x2 + d2 - 2.0 * xd)
    iota = jax.lax.broadcasted_iota(jnp.int32, scores.shape, 1)
    cols = []
    for _ in range(SPARSITY):
        m = jnp.max(scores, axis=1, keepdims=True)
        cand = jnp.where(scores == m, iota, 2 ** 30)
        ij = jnp.min(cand, axis=1, keepdims=True)
        cols.append(ij)
        scores = jnp.where(cand == ij, NEG_BIG, scores)
    idx_ref[...] = jnp.concatenate(cols, axis=1)


def _k3_body(l_ref, idx_ref, xb_ref, d_ref, m_ref, s_ref, g_ref, be_ref,
             mu_ref, var_ref, rep_ref, rout_ref, cnt_ref, sq_ref):
    i = pl.program_id(0)
    l = l_ref[...]
    sm = jnp.exp(l - m_ref[...]) / s_ref[...]
    a = g_ref[...] / jnp.sqrt(var_ref[...] + BN_EPS)
    repd = (sm - mu_ref[...]) * a + be_ref[...]

    iota = jax.lax.broadcasted_iota(jnp.int32, l.shape, 1)
    idx = idx_ref[...]
    mask = jnp.zeros_like(l)
    for j in range(SPARSITY):
        mask = mask + (iota == idx[:, j:j + 1]).astype(jnp.float32)
    rep = repd * mask
    rep_ref[...] = rep

    @pl.when(i == 0)
    def _():
        cnt_ref[...] = jnp.zeros_like(cnt_ref)
        sq_ref[...] = jnp.zeros_like(sq_ref)

    cnt_ref[...] += jnp.sum(mask, axis=0, keepdims=True)

    recon = jax.lax.dot_general(rep, d_ref[...], (((1,), (0,)), ((), ())),
                                preferred_element_type=jnp.float32)
    xb = xb_ref[...]
    rout_ref[...] = 2.0 * recon - xb
    diff = xb - recon
    sq_ref[...] += jnp.sum(diff * diff, axis=(0, 1), keepdims=True)


def _k4_body(cnt_ref, sq_ref, cc_ref, n_ref, loss_ref, perp_ref):
    n = n_ref[0, 0]
    avg = cnt_ref[...] / n
    p = avg / jnp.sum(avg, axis=(0, 1), keepdims=True)
    ent = -jnp.sum(p * jnp.log(p + EPS), axis=(0, 1), keepdims=True)
    perp_ref[...] = jnp.exp(ent)
    c = cnt_ref.shape[-1]
    del c
    loss_ref[...] = sq_ref[...] / n_ref[...] * (1.0 + cc_ref[...])


def kernel(x, dictionary, lin_w, lin_b, bn_gamma, bn_beta, bn_mean, bn_var,
           commitment_cost):
    B, C, H, W = x.shape
    N = B * H * W
    K = dictionary.shape[0]
    n_tiles = N // TN

    xf = jnp.transpose(x, (0, 2, 3, 1)).reshape(N, C)
    xb = x.reshape(N, C)

    row = lambda a: a.reshape(1, -1).astype(jnp.float32)

    logits, idx, m_col, s_col = pl.pallas_call(
        _k1_body,
        grid=(n_tiles,),
        in_specs=[
            pl.BlockSpec((TN, C), lambda i: (i, 0)),
            pl.BlockSpec((K, C), lambda i: (0, 0)),
            pl.BlockSpec((K, C), lambda i: (0, 0)),
            pl.BlockSpec((1, K), lambda i: (0, 0)),
        ],
        out_specs=[
            pl.BlockSpec((TN, K), lambda i: (i, 0)),
            pl.BlockSpec((TN, SPARSITY), lambda i: (i, 0)),
            pl.BlockSpec((1, K), lambda i: (0, 0)),
            pl.BlockSpec((1, K), lambda i: (0, 0)),
        ],
        out_shape=[
            jax.ShapeDtypeStruct((N, K), jnp.float32),
            jax.ShapeDtypeStruct((N, SPARSITY), jnp.int32),
            jax.ShapeDtypeStruct((1, K), jnp.float32),
            jax.ShapeDtypeStruct((1, K), jnp.float32),
        ],
    )(xf, lin_w, dictionary, row(lin_b))

    rep, rout, counts, sqsum = pl.pallas_call(
        _k3_body,
        grid=(n_tiles,),
        in_specs=[
            pl.BlockSpec((TN, K), lambda i: (i, 0)),
            pl.BlockSpec((TN, SPARSITY), lambda i: (i, 0)),
            pl.BlockSpec((TN, C), lambda i: (i, 0)),
            pl.BlockSpec((K, C), lambda i: (0, 0)),
            pl.BlockSpec((1, K), lambda i: (0, 0)),
            pl.BlockSpec((1, K), lambda i: (0, 0)),
            pl.BlockSpec((1, K), lambda i: (0, 0)),
            pl.BlockSpec((1, K), lambda i: (0, 0)),
            pl.BlockSpec((1, K), lambda i: (0, 0)),
            pl.BlockSpec((1, K), lambda i: (0, 0)),
        ],
        out_specs=[
            pl.BlockSpec((TN, K), lambda i: (i, 0)),
            pl.BlockSpec((TN, C), lambda i: (i, 0)),
            pl.BlockSpec((1, K), lambda i: (0, 0)),
            pl.BlockSpec((1, 1), lambda i: (0, 0)),
        ],
        out_shape=[
            jax.ShapeDtypeStruct((N, K), jnp.float32),
            jax.ShapeDtypeStruct((N, C), jnp.float32),
            jax.ShapeDtypeStruct((1, K), jnp.float32),
            jax.ShapeDtypeStruct((1, 1), jnp.float32),
        ],
    )(logits, idx, xb, dictionary, m_col, s_col, row(bn_gamma), row(bn_beta),
      row(bn_mean), row(bn_var))

    loss, perp = pl.pallas_call(
        _k4_body,
        in_specs=[
            pl.BlockSpec((1, K), lambda: (0, 0)),
            pl.BlockSpec((1, 1), lambda: (0, 0)),
            pl.BlockSpec((1, 1), lambda: (0, 0)),
            pl.BlockSpec((1, 1), lambda: (0, 0)),
        ],
        out_specs=[
            pl.BlockSpec((1, 1), lambda: (0, 0)),
            pl.BlockSpec((1, 1), lambda: (0, 0)),
        ],
        out_shape=[
            jax.ShapeDtypeStruct((1, 1), jnp.float32),
            jax.ShapeDtypeStruct((1, 1), jnp.float32),
        ],
    )(counts, sqsum, commitment_cost.reshape(1, 1).astype(jnp.float32),
      jnp.full((1, 1), float(N), jnp.float32))

    recon_loss = loss[0, 0] / jnp.float32(C)
    perplexity = perp[0, 0]
    recon_out = rout.reshape(B, C, H, W)
    return recon_loss, recon_out, perplexity, rep
